# Initial kernel scaffold; baseline (speedup 1.0000x reference)
import jax
import jax.numpy as jnp
from jax.experimental import pallas as pl


def kernel(x, edge_index, W1, b1, W2, b2):
    raise NotImplementedError("write your pallas kernel here")


# speedup vs baseline: 8.1906x; 8.1906x over previous
import functools

import jax
import jax.numpy as jnp
from jax import lax
from jax.experimental import pallas as pl
from jax.experimental.pallas import tpu as pltpu
from jax.experimental.pallas import tpu_sc as plsc

N = 10000
NP = 10240
E = 320000
D_IN = 128
DH = 256
DHH = 128
NT = 16
ROWS_PT = NP // NT

EB = 80
EDGES_PT = E // NT
NBLK = EDGES_PT // EB
DEG_EDGES_PT = E // (2 * NT)
DEG_NBLK = DEG_EDGES_PT // EB

_MESH = plsc.VectorSubcoreMesh(core_axis_name="c", subcore_axis_name="s")


DEGW = 128


@functools.partial(
    pl.kernel,
    out_type=jax.ShapeDtypeStruct((2 * NP, DEGW), jnp.float32),
    mesh=_MESH,
    scratch_types=[
        pltpu.VMEM_SHARED((NP, DEGW), jnp.float32),
        pltpu.VMEM((EB,), jnp.int32),
        pltpu.VMEM((EB, DEGW), jnp.float32),
    ],
)
def _deg_kernel(dst_hbm, ones_hbm, zeros_hbm, deg_out, deg_sh, idx_v, ones_v):
    c = lax.axis_index("c")
    s = lax.axis_index("s")
    r0 = s * ROWS_PT
    pltpu.sync_copy(zeros_hbm.at[pl.ds(r0, ROWS_PT)], deg_sh.at[pl.ds(r0, ROWS_PT)])
    pltpu.sync_copy(ones_hbm, ones_v)
    plsc.subcore_barrier()
    base = (c * NT + s) * DEG_EDGES_PT

    def body(i, carry):
        e0 = base + i * EB
        pltpu.sync_copy(dst_hbm.at[pl.ds(e0, EB)], idx_v)
        pltpu.sync_copy(ones_v, deg_sh.at[idx_v], add=True)
        return carry

    lax.fori_loop(0, DEG_NBLK, body, 0)
    plsc.subcore_barrier()
    pltpu.sync_copy(deg_sh.at[pl.ds(r0, ROWS_PT)],
                    deg_out.at[pl.ds(c * NP + r0, ROWS_PT)])


@functools.partial(
    pl.kernel,
    out_type=jax.ShapeDtypeStruct((2 * NP, DHH), jnp.float32),
    mesh=_MESH,
    scratch_types=[
        pltpu.VMEM_SHARED((NP, DHH), jnp.float32),
        pltpu.VMEM((EB,), jnp.int32),
        pltpu.VMEM((EB,), jnp.int32),
        pltpu.VMEM((EB, DHH), jnp.float32),
        pltpu.SemaphoreType.DMA,
    ],
)
def _prop_kernel(y_hbm, src_hbm, dst_hbm, out_hbm, acc_sh, src_v, dst_v, rows_v, sem):
    c = lax.axis_index("c")
    s = lax.axis_index("s")
    r0 = s * ROWS_PT
    off = c * NP
    pltpu.sync_copy(y_hbm.at[pl.ds(off + r0, ROWS_PT)], acc_sh.at[pl.ds(r0, ROWS_PT)])
    plsc.subcore_barrier()
    base = s * EDGES_PT

    def body(i, carry):
        e0 = base + i * EB
        pltpu.sync_copy(src_hbm.at[pl.ds(e0, EB)], src_v)
        pltpu.sync_copy(dst_hbm.at[pl.ds(e0, EB)], dst_v)
        for k in range(EB // 16):
            sl = pl.ds(k * 16, 16)
            src_v[sl] = src_v[sl] + off
        pltpu.async_copy(y_hbm.at[src_v], rows_v, sem).wait()
        pltpu.sync_copy(rows_v, acc_sh.at[dst_v], add=True)
        return carry

    lax.fori_loop(0, NBLK, body, 0)
    plsc.subcore_barrier()
    pltpu.sync_copy(acc_sh.at[pl.ds(r0, ROWS_PT)],
                    out_hbm.at[pl.ds(off + r0, ROWS_PT)])


BM = 2048
GRID = NP // BM


def _dis(deg_ref):
    d = deg_ref[:, 0:1] + deg_ref[:, 1:2] + 1.0
    return lax.rsqrt(d)


def _elu(v):
    return jnp.where(v > 0, v, jnp.exp(v) - 1.0)


def _mm1_body(x_ref, w_ref, deg_ref, o_ref):
    dis = _dis(deg_ref)
    xw = jnp.dot(x_ref[...], w_ref[...], preferred_element_type=jnp.float32,
                 precision=lax.Precision.HIGHEST)
    y = dis * xw
    o_ref[0] = y[:, 0:DHH]
    o_ref[1] = y[:, DHH:DH]


def _mm2_body(acc_ref, deg_ref, b1_ref, w2_ref, o_ref):
    dis = _dis(deg_ref)
    h0 = _elu(dis * acc_ref[0] + b1_ref[:, 0:DHH])
    h1 = _elu(dis * acc_ref[1] + b1_ref[:, DHH:DH])
    y = dis * (jnp.dot(h0, w2_ref[0:DHH, :], preferred_element_type=jnp.float32,
                       precision=lax.Precision.HIGHEST)
               + jnp.dot(h1, w2_ref[DHH:DH, :], preferred_element_type=jnp.float32,
                         precision=lax.Precision.HIGHEST))
    o_ref[0] = y[:, 0:DHH]
    o_ref[1] = y[:, DHH:DH]


def _out_body(acc_ref, deg_ref, b2_ref, o_ref):
    dis = _dis(deg_ref)
    o_ref[:, 0:DHH] = _elu(dis * acc_ref[0] + b2_ref[:, 0:DHH])
    o_ref[:, DHH:DH] = _elu(dis * acc_ref[1] + b2_ref[:, DHH:DH])


_half_spec = pl.BlockSpec((2, BM, DHH), lambda i: (0, i, 0))
_deg_spec = pl.BlockSpec((BM, 2), lambda i: (i, 0))

_mm1 = pl.pallas_call(
    _mm1_body,
    grid=(GRID,),
    in_specs=[pl.BlockSpec((BM, D_IN), lambda i: (i, 0)),
              pl.BlockSpec((D_IN, DH), lambda i: (0, 0)),
              _deg_spec],
    out_specs=_half_spec,
    out_shape=jax.ShapeDtypeStruct((2, NP, DHH), jnp.float32))

_mm2 = pl.pallas_call(
    _mm2_body,
    grid=(GRID,),
    in_specs=[_half_spec, _deg_spec,
              pl.BlockSpec((1, DH), lambda i: (0, 0)),
              pl.BlockSpec((DH, DH), lambda i: (0, 0))],
    out_specs=_half_spec,
    out_shape=jax.ShapeDtypeStruct((2, NP, DHH), jnp.float32))

_out = pl.pallas_call(
    _out_body,
    grid=(GRID,),
    in_specs=[_half_spec, _deg_spec,
              pl.BlockSpec((1, DH), lambda i: (0, 0))],
    out_specs=pl.BlockSpec((BM, DH), lambda i: (i, 0)),
    out_shape=jax.ShapeDtypeStruct((NP, DH), jnp.float32))


def kernel(x, edge_index, W1, b1, W2, b2):
    src = edge_index[0].astype(jnp.int32)
    dst = edge_index[1].astype(jnp.int32)
    xp = jnp.pad(x.astype(jnp.float32), ((0, NP - N), (0, 0)))
    ones = jnp.ones((EB, DEGW), jnp.float32)
    zeros = jnp.zeros((NP, DEGW), jnp.float32)

    deg = _deg_kernel(dst, ones, zeros).reshape(2, NP, DEGW)[:, :, 0].T
    y1 = _mm1(xp, W1, deg)
    acc1 = _prop_kernel(y1.reshape(2 * NP, DHH), src, dst)
    y2 = _mm2(acc1.reshape(2, NP, DHH), deg, b1.reshape(1, DH), W2)
    acc2 = _prop_kernel(y2.reshape(2 * NP, DHH), src, dst)
    out = _out(acc2.reshape(2, NP, DHH), deg, b2.reshape(1, DH))
    return out[:N]
